# Initial kernel scaffold; baseline (speedup 1.0000x reference)
import functools

import jax
import jax.numpy as jnp
from jax import lax
from jax.experimental import pallas as pl
from jax.experimental.pallas import tpu as pltpu
from jax.experimental.pallas import tpu_sc as plsc

LANE = 128
NSUB = 16
VEC = 16
BN = 2048


def _softplus(v):
    return jnp.maximum(v, 0.0) + jnp.log1p(jnp.exp(-jnp.abs(v)))


def _silu(v):
    return v / (1.0 + jnp.exp(-v))


def _make_sc_gather(rows_pad, rpt, b_pad):
    mesh = plsc.VectorSubcoreMesh(core_axis_name="c", subcore_axis_name="s",
                                  num_cores=1)

    @functools.partial(
        pl.kernel,
        out_type=jax.ShapeDtypeStruct((rows_pad, LANE), jnp.float32),
        mesh=mesh,
        scratch_types=[
            pltpu.VMEM((rpt, LANE), jnp.int32),
            pltpu.VMEM((rpt, LANE), jnp.float32),
            pltpu.SemaphoreType.DMA,
        ],
    )
    def sc_gather(e_hbm, seg_hbm, out_hbm, idx_v, val_v, sem):
        wid = lax.axis_index("s")
        r0 = wid * rpt
        pltpu.sync_copy(seg_hbm.at[pl.ds(r0, rpt)], idx_v)

        def fire(j, c):
            pltpu.async_copy(e_hbm.at[idx_v.at[j]], val_v.at[j], sem)
            return c

        lax.fori_loop(0, rpt, fire, 0)

        def drain(j, c):
            pltpu.make_async_copy(e_hbm.at[idx_v.at[j]], val_v.at[j],
                                  sem).wait()
            return c

        lax.fori_loop(0, rpt, drain, 0)
        pltpu.sync_copy(val_v, out_hbm.at[pl.ds(r0, rpt)])

    return sc_gather


def _make_sc_segnorm(rows_pad, rpt, b_pad):
    spt = b_pad // NSUB
    mesh = plsc.VectorSubcoreMesh(core_axis_name="c", subcore_axis_name="s",
                                  num_cores=1)

    @functools.partial(
        pl.kernel,
        out_type=jax.ShapeDtypeStruct((rows_pad, LANE), jnp.float32),
        mesh=mesh,
        scratch_types=[
            pltpu.VMEM((rpt, LANE), jnp.int32),
            pltpu.VMEM((rpt, LANE), jnp.float32),
            pltpu.VMEM((rpt, LANE), jnp.float32),
            pltpu.VMEM((LANE,), jnp.float32),
            pltpu.VMEM((b_pad // NSUB,), jnp.float32),
            pltpu.VMEM((b_pad // NSUB,), jnp.float32),
            pltpu.VMEM((b_pad // NSUB,), jnp.float32),
            pltpu.VMEM((b_pad // NSUB,), jnp.float32),
            pltpu.VMEM_SHARED((b_pad,), jnp.float32),
            pltpu.VMEM_SHARED((b_pad,), jnp.float32),
            pltpu.VMEM_SHARED((b_pad,), jnp.float32),
            pltpu.SemaphoreType.DMA,
        ],
    )
    def sc_segnorm(e_hbm, seg_hbm, et_hbm, out_hbm, idx_v, et_v, d_v, ones_v,
                   zero_v, pred_v, cnt_v, eseg_v, pred_sh, cnt_sh, d_sh, sem):
        wid = lax.axis_index("s")
        r0 = wid * rpt
        b0 = wid * spt

        pltpu.sync_copy(seg_hbm.at[pl.ds(r0, rpt)], idx_v)
        pltpu.sync_copy(et_hbm.at[pl.ds(r0, rpt)], et_v)

        for i in range(LANE // VEC):
            ones_v[pl.ds(i * VEC, VEC)] = jnp.full((VEC,), 1.0, jnp.float32)
        for i in range(spt // VEC):
            zero_v[pl.ds(i * VEC, VEC)] = jnp.zeros((VEC,), jnp.float32)
        pltpu.sync_copy(zero_v, pred_sh.at[pl.ds(b0, spt)])
        pltpu.sync_copy(zero_v, cnt_sh.at[pl.ds(b0, spt)])
        plsc.subcore_barrier()

        def fire(j, c):
            pltpu.async_copy(et_v.at[j], pred_sh.at[idx_v.at[j]], sem,
                             add=True)
            pltpu.async_copy(ones_v, cnt_sh.at[idx_v.at[j]], sem, add=True)
            return c

        lax.fori_loop(0, rpt, fire, 0)

        def drain(j, c):
            pltpu.make_async_copy(et_v.at[j], pred_sh.at[idx_v.at[j]],
                                  sem).wait()
            pltpu.make_async_copy(ones_v, cnt_sh.at[idx_v.at[j]],
                                  sem).wait()
            return c

        lax.fori_loop(0, rpt, drain, 0)
        plsc.subcore_barrier()

        pltpu.sync_copy(pred_sh.at[pl.ds(b0, spt)], pred_v)
        pltpu.sync_copy(cnt_sh.at[pl.ds(b0, spt)], cnt_v)
        pltpu.sync_copy(e_hbm.at[pl.ds(b0, spt)], eseg_v)
        for i in range(spt // VEC):
            sl = pl.ds(i * VEC, VEC)
            pred_v[sl] = (eseg_v[sl] - pred_v[sl]) / cnt_v[sl]
        pltpu.sync_copy(pred_v, d_sh.at[pl.ds(b0, spt)])
        plsc.subcore_barrier()

        def fire_g(j, c):
            pltpu.async_copy(d_sh.at[idx_v.at[j]], d_v.at[j], sem)
            return c

        lax.fori_loop(0, rpt, fire_g, 0)

        def drain_g(j, c):
            pltpu.make_async_copy(d_sh.at[idx_v.at[j]], d_v.at[j], sem).wait()
            return c

        lax.fori_loop(0, rpt, drain_g, 0)
        pltpu.sync_copy(d_v, out_hbm.at[pl.ds(r0, rpt)])

    return sc_segnorm


def _tc1_body(x_ref, e_ref, wq_ref, bq_ref, wk_ref, bk_ref, out_ref):
    q = jnp.dot(x_ref[...], wq_ref[...],
                preferred_element_type=jnp.float32) + bq_ref[...]
    k = jnp.abs(e_ref[...]) * wk_ref[...] + bk_ref[...]
    sp = _softplus(k * q)
    out_ref[...] = jnp.sum(sp, axis=1, keepdims=True)


def _tc2_body(x_ref, e_ref, d_ref, wq_ref, bq_ref, wk_ref, bk_ref,
              w1_ref, b1_ref, w2_ref, b2_ref, out_ref):
    f = x_ref.shape[1]
    q = jnp.dot(x_ref[...], wq_ref[...],
                preferred_element_type=jnp.float32) + bq_ref[...]
    k = jnp.abs(e_ref[...]) * wk_ref[...] + bk_ref[...]
    embed = _softplus(k * q)
    scale = embed + d_ref[...] * (1.0 / f)
    h = jnp.dot(_silu(scale), w1_ref[...],
                preferred_element_type=jnp.float32) + b1_ref[...]
    h = jnp.dot(_silu(h), w2_ref[...],
                preferred_element_type=jnp.float32) + b2_ref[...]
    out_ref[...] = scale + h


def _row_spec(bn, w):
    return pl.BlockSpec((bn, w), lambda i: (i, 0))


def _full_spec(shape):
    return pl.BlockSpec(shape, lambda i: (0, 0))


def kernel(x, E, num_batch, batch_seg, W_q, b_q, W_k, b_k, W_r1, b_r1,
           W_r2, b_r2, eps):
    n, f = x.shape
    b = E.shape[0]

    rows = -(-n // LANE)
    rpt = -(-rows // NSUB)
    rows_pad = rpt * NSUB
    n_pad = rows_pad * LANE
    b_pad = -(-(b + 1) // (NSUB * VEC)) * (NSUB * VEC)

    e32 = E.astype(jnp.float32)
    e_pad = jnp.pad(e32, (0, b_pad - b))
    seg2d = jnp.pad(batch_seg.astype(jnp.int32), (0, n_pad - n),
                    constant_values=b).reshape(rows_pad, LANE)

    wq_t = W_q.T
    bq_row = b_q.reshape(1, f)
    wk_row = W_k.reshape(1, f)
    bk_row = b_k.reshape(1, f)
    w1_t = W_r1.T
    b1_row = b_r1.reshape(1, f)
    w2_t = W_r2.T
    b2_row = b_r2.reshape(1, f)

    e2d = _make_sc_gather(rows_pad, rpt, b_pad)(e_pad, seg2d)
    e_col = e2d.reshape(n_pad)[:n].reshape(n, 1)

    grid = (-(-n // BN),)
    et_col = pl.pallas_call(
        _tc1_body,
        grid=grid,
        in_specs=[_row_spec(BN, f), _row_spec(BN, 1), _full_spec((f, f)),
                  _full_spec((1, f)), _full_spec((1, f)), _full_spec((1, f))],
        out_specs=_row_spec(BN, 1),
        out_shape=jax.ShapeDtypeStruct((n, 1), jnp.float32),
    )(x, e_col, wq_t, bq_row, wk_row, bk_row)

    et2d = jnp.pad(et_col.reshape(n), (0, n_pad - n)).reshape(rows_pad, LANE)
    d2d = _make_sc_segnorm(rows_pad, rpt, b_pad)(e_pad, seg2d, et2d)
    d_col = d2d.reshape(n_pad)[:n].reshape(n, 1)

    out = pl.pallas_call(
        _tc2_body,
        grid=grid,
        in_specs=[_row_spec(BN, f), _row_spec(BN, 1), _row_spec(BN, 1),
                  _full_spec((f, f)), _full_spec((1, f)), _full_spec((1, f)),
                  _full_spec((1, f)), _full_spec((f, f)), _full_spec((1, f)),
                  _full_spec((f, f)), _full_spec((1, f))],
        out_specs=_row_spec(BN, f),
        out_shape=jax.ShapeDtypeStruct((n, f), jnp.float32),
    )(x, e_col, d_col, wq_t, bq_row, wk_row, bk_row, w1_t, b1_row,
      w2_t, b2_row)
    return out


# speedup vs baseline: 3.4636x; 3.4636x over previous
import functools

import jax
import jax.numpy as jnp
from jax import lax
from jax.experimental import pallas as pl
from jax.experimental.pallas import tpu as pltpu
from jax.experimental.pallas import tpu_sc as plsc

LANE = 128
NSUB = 16
VEC = 16
BN = 2048


def _softplus(v):
    return jnp.maximum(v, 0.0) + jnp.log1p(jnp.exp(-jnp.abs(v)))


def _silu(v):
    return v / (1.0 + jnp.exp(-v))


def _make_sc_gather(rows_pad, rpt, b_pad):
    mesh = plsc.VectorSubcoreMesh(core_axis_name="c", subcore_axis_name="s",
                                  num_cores=1)

    @functools.partial(
        pl.kernel,
        out_type=jax.ShapeDtypeStruct((rows_pad, LANE), jnp.float32),
        mesh=mesh,
        scratch_types=[
            pltpu.VMEM((rpt, LANE), jnp.int32),
            pltpu.VMEM((rpt, LANE), jnp.float32),
            pltpu.SemaphoreType.DMA,
        ],
    )
    def sc_gather(e_hbm, seg_hbm, out_hbm, idx_v, val_v, sem):
        wid = lax.axis_index("s")
        r0 = wid * rpt
        pltpu.sync_copy(seg_hbm.at[pl.ds(r0, rpt)], idx_v)

        def fire(j, c):
            pltpu.async_copy(e_hbm.at[idx_v.at[j]], val_v.at[j], sem)
            return c

        lax.fori_loop(0, rpt, fire, 0)

        def drain(j, c):
            pltpu.make_async_copy(e_hbm.at[idx_v.at[j]], val_v.at[j],
                                  sem).wait()
            return c

        lax.fori_loop(0, rpt, drain, 0)
        pltpu.sync_copy(val_v, out_hbm.at[pl.ds(r0, rpt)])

    return sc_gather


def _make_sc_segnorm(rows_pad, rpt, b_pad):
    spt = b_pad // NSUB
    mesh = plsc.VectorSubcoreMesh(core_axis_name="c", subcore_axis_name="s",
                                  num_cores=1)

    @functools.partial(
        pl.kernel,
        out_type=jax.ShapeDtypeStruct((rows_pad, LANE), jnp.float32),
        mesh=mesh,
        scratch_types=[
            pltpu.VMEM((rpt, LANE), jnp.int32),
            pltpu.VMEM((rpt, LANE), jnp.float32),
            pltpu.VMEM((rpt, LANE), jnp.float32),
            pltpu.VMEM((LANE,), jnp.float32),
            pltpu.VMEM((b_pad // NSUB,), jnp.float32),
            pltpu.VMEM((b_pad // NSUB,), jnp.float32),
            pltpu.VMEM((b_pad // NSUB,), jnp.float32),
            pltpu.VMEM((b_pad // NSUB,), jnp.float32),
            pltpu.VMEM_SHARED((b_pad,), jnp.float32),
            pltpu.VMEM_SHARED((b_pad,), jnp.float32),
            pltpu.VMEM_SHARED((b_pad,), jnp.float32),
            pltpu.SemaphoreType.DMA,
        ],
    )
    def sc_segnorm(e_hbm, seg_hbm, et_hbm, out_hbm, idx_v, et_v, d_v, ones_v,
                   zero_v, pred_v, cnt_v, eseg_v, pred_sh, cnt_sh, d_sh, sem):
        wid = lax.axis_index("s")
        r0 = wid * rpt
        b0 = wid * spt

        pltpu.sync_copy(seg_hbm.at[pl.ds(r0, rpt)], idx_v)
        pltpu.sync_copy(et_hbm.at[pl.ds(r0, rpt)], et_v)

        for i in range(LANE // VEC):
            ones_v[pl.ds(i * VEC, VEC)] = jnp.full((VEC,), 1.0, jnp.float32)
        for i in range(spt // VEC):
            zero_v[pl.ds(i * VEC, VEC)] = jnp.zeros((VEC,), jnp.float32)
        pltpu.sync_copy(zero_v, pred_sh.at[pl.ds(b0, spt)])
        pltpu.sync_copy(zero_v, cnt_sh.at[pl.ds(b0, spt)])
        plsc.subcore_barrier()

        def fire(j, c):
            pltpu.async_copy(et_v.at[j], pred_sh.at[idx_v.at[j]], sem,
                             add=True)
            pltpu.async_copy(ones_v, cnt_sh.at[idx_v.at[j]], sem, add=True)
            return c

        lax.fori_loop(0, rpt, fire, 0)

        def drain(j, c):
            pltpu.make_async_copy(et_v.at[j], pred_sh.at[idx_v.at[j]],
                                  sem).wait()
            pltpu.make_async_copy(ones_v, cnt_sh.at[idx_v.at[j]],
                                  sem).wait()
            return c

        lax.fori_loop(0, rpt, drain, 0)
        plsc.subcore_barrier()

        pltpu.sync_copy(pred_sh.at[pl.ds(b0, spt)], pred_v)
        pltpu.sync_copy(cnt_sh.at[pl.ds(b0, spt)], cnt_v)
        pltpu.sync_copy(e_hbm.at[pl.ds(b0, spt)], eseg_v)
        for i in range(spt // VEC):
            sl = pl.ds(i * VEC, VEC)
            pred_v[sl] = (eseg_v[sl] - pred_v[sl]) / cnt_v[sl]
        pltpu.sync_copy(pred_v, d_sh.at[pl.ds(b0, spt)])
        plsc.subcore_barrier()

        def fire_g(j, c):
            pltpu.async_copy(d_sh.at[idx_v.at[j]], d_v.at[j], sem)
            return c

        lax.fori_loop(0, rpt, fire_g, 0)

        def drain_g(j, c):
            pltpu.make_async_copy(d_sh.at[idx_v.at[j]], d_v.at[j], sem).wait()
            return c

        lax.fori_loop(0, rpt, drain_g, 0)
        pltpu.sync_copy(d_v, out_hbm.at[pl.ds(r0, rpt)])

    return sc_segnorm


def _tc1_body(x_ref, e_ref, wq_ref, bq_ref, wk_ref, bk_ref, out_ref):
    q = jnp.dot(x_ref[...], wq_ref[...],
                preferred_element_type=jnp.float32) + bq_ref[...]
    k = jnp.abs(e_ref[...]) * wk_ref[...] + bk_ref[...]
    sp = _softplus(k * q)
    out_ref[...] = jnp.sum(sp, axis=1, keepdims=True)


def _tc2_body(x_ref, e_ref, d_ref, wq_ref, bq_ref, wk_ref, bk_ref,
              w1_ref, b1_ref, w2_ref, b2_ref, out_ref):
    f = x_ref.shape[1]
    q = jnp.dot(x_ref[...], wq_ref[...],
                preferred_element_type=jnp.float32) + bq_ref[...]
    k = jnp.abs(e_ref[...]) * wk_ref[...] + bk_ref[...]
    embed = _softplus(k * q)
    scale = embed + d_ref[...] * (1.0 / f)
    h = jnp.dot(_silu(scale), w1_ref[...],
                preferred_element_type=jnp.float32) + b1_ref[...]
    h = jnp.dot(_silu(h), w2_ref[...],
                preferred_element_type=jnp.float32) + b2_ref[...]
    out_ref[...] = scale + h


def _row_spec(bn, w):
    return pl.BlockSpec((bn, w), lambda i: (i, 0))


def _full_spec(shape):
    return pl.BlockSpec(shape, lambda i: (0, 0))


def kernel(x, E, num_batch, batch_seg, W_q, b_q, W_k, b_k, W_r1, b_r1,
           W_r2, b_r2, eps):
    n, f = x.shape
    b = E.shape[0]

    rows = -(-n // LANE)
    rpt = -(--(-rows // NSUB) // 8) * 8
    rows_pad = rpt * NSUB
    n_pad = rows_pad * LANE
    b_pad = -(-(b + 1) // (NSUB * VEC)) * (NSUB * VEC)

    e32 = E.astype(jnp.float32)
    e_pad = jnp.pad(e32, (0, b_pad - b))
    seg2d = jnp.pad(batch_seg.astype(jnp.int32), (0, n_pad - n),
                    constant_values=b).reshape(rows_pad, LANE)

    wq_t = W_q.T
    bq_row = b_q.reshape(1, f)
    wk_row = W_k.reshape(1, f)
    bk_row = b_k.reshape(1, f)
    w1_t = W_r1.T
    b1_row = b_r1.reshape(1, f)
    w2_t = W_r2.T
    b2_row = b_r2.reshape(1, f)

    e2d = _make_sc_gather(rows_pad, rpt, b_pad)(e_pad, seg2d)
    e_col = e2d.reshape(n_pad)[:n].reshape(n, 1)

    grid = (-(-n // BN),)
    et_col = pl.pallas_call(
        _tc1_body,
        grid=grid,
        in_specs=[_row_spec(BN, f), _row_spec(BN, 1), _full_spec((f, f)),
                  _full_spec((1, f)), _full_spec((1, f)), _full_spec((1, f))],
        out_specs=_row_spec(BN, 1),
        out_shape=jax.ShapeDtypeStruct((n, 1), jnp.float32),
    )(x, e_col, wq_t, bq_row, wk_row, bk_row)

    et2d = jnp.pad(et_col.reshape(n), (0, n_pad - n)).reshape(rows_pad, LANE)
    d2d = _make_sc_segnorm(rows_pad, rpt, b_pad)(e_pad, seg2d, et2d)
    d_col = d2d.reshape(n_pad)[:n].reshape(n, 1)

    out = pl.pallas_call(
        _tc2_body,
        grid=grid,
        in_specs=[_row_spec(BN, f), _row_spec(BN, 1), _row_spec(BN, 1),
                  _full_spec((f, f)), _full_spec((1, f)), _full_spec((1, f)),
                  _full_spec((1, f)), _full_spec((f, f)), _full_spec((1, f)),
                  _full_spec((f, f)), _full_spec((1, f))],
        out_specs=_row_spec(BN, f),
        out_shape=jax.ShapeDtypeStruct((n, f), jnp.float32),
    )(x, e_col, d_col, wq_t, bq_row, wk_row, bk_row, w1_t, b1_row,
      w2_t, b2_row)
    return out


# speedup vs baseline: 5.0508x; 1.4583x over previous
import functools

import jax
import jax.numpy as jnp
from jax import lax
from jax.experimental import pallas as pl
from jax.experimental.pallas import tpu as pltpu
from jax.experimental.pallas import tpu_sc as plsc

LANE = 128
NSUB = 16
VEC = 16
BN = 2048


def _softplus(v):
    return jnp.maximum(v, 0.0) + jnp.log1p(jnp.exp(-jnp.abs(v)))


def _silu(v):
    return v / (1.0 + jnp.exp(-v))


def _make_sc_gather(rows_pad, rpt, b_pad):
    mesh = plsc.VectorSubcoreMesh(core_axis_name="c", subcore_axis_name="s",
                                  num_cores=1)

    @functools.partial(
        pl.kernel,
        out_type=jax.ShapeDtypeStruct((rows_pad, LANE), jnp.float32),
        mesh=mesh,
        scratch_types=[
            pltpu.VMEM((rpt, LANE), jnp.int32),
            pltpu.VMEM((rpt, LANE), jnp.float32),
            pltpu.SemaphoreType.DMA,
        ],
    )
    def sc_gather(e_hbm, seg_hbm, out_hbm, idx_v, val_v, sem):
        wid = lax.axis_index("s")
        r0 = wid * rpt
        pltpu.sync_copy(seg_hbm.at[pl.ds(r0, rpt)], idx_v)

        def fire(j, c):
            pltpu.async_copy(e_hbm.at[idx_v.at[j]], val_v.at[j], sem)
            return c

        lax.fori_loop(0, rpt, fire, 0)

        def drain(j, c):
            pltpu.make_async_copy(e_hbm.at[idx_v.at[j]], val_v.at[j],
                                  sem).wait()
            return c

        lax.fori_loop(0, rpt, drain, 0)
        pltpu.sync_copy(val_v, out_hbm.at[pl.ds(r0, rpt)])

    return sc_gather


def _make_sc_segnorm(rows_pad, rpt, b_pad):
    spt = b_pad // NSUB
    mesh = plsc.VectorSubcoreMesh(core_axis_name="c", subcore_axis_name="s",
                                  num_cores=1)

    @functools.partial(
        pl.kernel,
        out_type=jax.ShapeDtypeStruct((rows_pad, LANE), jnp.float32),
        mesh=mesh,
        scratch_types=[
            pltpu.VMEM((rpt, LANE), jnp.int32),
            pltpu.VMEM((rpt, LANE), jnp.float32),
            pltpu.VMEM((rpt, LANE), jnp.float32),
            pltpu.VMEM((LANE,), jnp.float32),
            pltpu.VMEM((b_pad // NSUB,), jnp.float32),
            pltpu.VMEM((b_pad // NSUB,), jnp.float32),
            pltpu.VMEM((b_pad // NSUB,), jnp.float32),
            pltpu.VMEM((b_pad // NSUB,), jnp.float32),
            pltpu.VMEM_SHARED((b_pad,), jnp.float32),
            pltpu.VMEM_SHARED((b_pad,), jnp.float32),
            pltpu.VMEM_SHARED((b_pad,), jnp.float32),
            pltpu.SemaphoreType.DMA,
        ],
    )
    def sc_segnorm(e_hbm, seg_hbm, et_hbm, out_hbm, idx_v, et_v, d_v, ones_v,
                   zero_v, pred_v, cnt_v, eseg_v, pred_sh, cnt_sh, d_sh, sem):
        wid = lax.axis_index("s")
        r0 = wid * rpt
        b0 = wid * spt

        pltpu.sync_copy(seg_hbm.at[pl.ds(r0, rpt)], idx_v)
        pltpu.sync_copy(et_hbm.at[pl.ds(r0, rpt)], et_v)

        for i in range(LANE // VEC):
            ones_v[pl.ds(i * VEC, VEC)] = jnp.full((VEC,), 1.0, jnp.float32)
        for i in range(spt // VEC):
            zero_v[pl.ds(i * VEC, VEC)] = jnp.zeros((VEC,), jnp.float32)
        pltpu.sync_copy(zero_v, pred_sh.at[pl.ds(b0, spt)])
        pltpu.sync_copy(zero_v, cnt_sh.at[pl.ds(b0, spt)])
        plsc.subcore_barrier()

        def fire(j, c):
            pltpu.async_copy(et_v.at[j], pred_sh.at[idx_v.at[j]], sem,
                             add=True)
            pltpu.async_copy(ones_v, cnt_sh.at[idx_v.at[j]], sem, add=True)
            return c

        lax.fori_loop(0, rpt, fire, 0)

        def drain(j, c):
            pltpu.make_async_copy(et_v.at[j], pred_sh.at[idx_v.at[j]],
                                  sem).wait()
            pltpu.make_async_copy(ones_v, cnt_sh.at[idx_v.at[j]],
                                  sem).wait()
            return c

        lax.fori_loop(0, rpt, drain, 0)
        plsc.subcore_barrier()

        pltpu.sync_copy(pred_sh.at[pl.ds(b0, spt)], pred_v)
        pltpu.sync_copy(cnt_sh.at[pl.ds(b0, spt)], cnt_v)
        pltpu.sync_copy(e_hbm.at[pl.ds(b0, spt)], eseg_v)
        for i in range(spt // VEC):
            sl = pl.ds(i * VEC, VEC)
            pred_v[sl] = (eseg_v[sl] - pred_v[sl]) / cnt_v[sl]
        pltpu.sync_copy(pred_v, d_sh.at[pl.ds(b0, spt)])
        plsc.subcore_barrier()

        def fire_g(j, c):
            pltpu.async_copy(d_sh.at[idx_v.at[j]], d_v.at[j], sem)
            return c

        lax.fori_loop(0, rpt, fire_g, 0)

        def drain_g(j, c):
            pltpu.make_async_copy(d_sh.at[idx_v.at[j]], d_v.at[j], sem).wait()
            return c

        lax.fori_loop(0, rpt, drain_g, 0)
        pltpu.sync_copy(d_v, out_hbm.at[pl.ds(r0, rpt)])

    return sc_segnorm


def _lane_to_col(v2d, ident):
    return jnp.sum(v2d[:, None, :] * ident[None, :, :], axis=2,
                   keepdims=True)


def _col_to_lane(c3, ident):
    return jnp.sum(c3 * ident[None, :, :], axis=1)


def _tc1_body(x_ref, e_ref, wq_ref, bq_ref, wk_ref, bk_ref, id_ref,
              out_ref):
    f = x_ref.shape[1]
    r = x_ref.shape[0] // f
    q = jnp.dot(x_ref[...], wq_ref[...],
                preferred_element_type=jnp.float32) + bq_ref[...]
    q3 = q.reshape(r, f, f)
    e_col = _lane_to_col(e_ref[...], id_ref[...])
    k3 = jnp.abs(e_col) * wk_ref[...].reshape(1, 1, f) \
        + bk_ref[...].reshape(1, 1, f)
    sp = _softplus(k3 * q3)
    et_col = jnp.sum(sp, axis=2, keepdims=True)
    out_ref[...] = _col_to_lane(et_col, id_ref[...])


def _tc2_body(x_ref, e_ref, d_ref, wq_ref, bq_ref, wk_ref, bk_ref,
              w1_ref, b1_ref, w2_ref, b2_ref, id_ref, out_ref):
    f = x_ref.shape[1]
    r = x_ref.shape[0] // f
    q = jnp.dot(x_ref[...], wq_ref[...],
                preferred_element_type=jnp.float32) + bq_ref[...]
    q3 = q.reshape(r, f, f)
    e_col = _lane_to_col(e_ref[...], id_ref[...])
    d_col = _lane_to_col(d_ref[...], id_ref[...])
    k3 = jnp.abs(e_col) * wk_ref[...].reshape(1, 1, f) \
        + bk_ref[...].reshape(1, 1, f)
    embed = _softplus(k3 * q3)
    scale = embed + d_col * (1.0 / f)
    h = jnp.dot(_silu(scale).reshape(r * f, f), w1_ref[...],
                preferred_element_type=jnp.float32) + b1_ref[...]
    h = jnp.dot(_silu(h), w2_ref[...],
                preferred_element_type=jnp.float32) + b2_ref[...]
    out_ref[...] = scale.reshape(r * f, f) + h


def _row_spec(bn, w):
    return pl.BlockSpec((bn, w), lambda i: (i, 0))


def _full_spec(shape):
    return pl.BlockSpec(shape, lambda i: (0, 0))


def kernel(x, E, num_batch, batch_seg, W_q, b_q, W_k, b_k, W_r1, b_r1,
           W_r2, b_r2, eps):
    n, f = x.shape
    b = E.shape[0]

    rows = -(-n // LANE)
    rpt = -(--(-rows // NSUB) // 8) * 8
    rows_pad = rpt * NSUB
    n_pad = rows_pad * LANE
    b_pad = -(-(b + 1) // (NSUB * VEC)) * (NSUB * VEC)

    e32 = E.astype(jnp.float32)
    e_pad = jnp.pad(e32, (0, b_pad - b))
    seg2d = jnp.pad(batch_seg.astype(jnp.int32), (0, n_pad - n),
                    constant_values=b).reshape(rows_pad, LANE)

    wq_t = W_q.T
    bq_row = b_q.reshape(1, f)
    wk_row = W_k.reshape(1, f)
    bk_row = b_k.reshape(1, f)
    w1_t = W_r1.T
    b1_row = b_r1.reshape(1, f)
    w2_t = W_r2.T
    b2_row = b_r2.reshape(1, f)

    ident = jnp.eye(f, dtype=jnp.float32)
    rb = BN // LANE

    e2d = _make_sc_gather(rows_pad, rpt, b_pad)(e_pad, seg2d)

    grid = (-(-n // BN),)
    et2d = pl.pallas_call(
        _tc1_body,
        grid=grid,
        in_specs=[_row_spec(BN, f), _row_spec(rb, LANE), _full_spec((f, f)),
                  _full_spec((1, f)), _full_spec((1, f)), _full_spec((1, f)),
                  _full_spec((f, f))],
        out_specs=_row_spec(rb, LANE),
        out_shape=jax.ShapeDtypeStruct((rows_pad, LANE), jnp.float32),
    )(x, e2d, wq_t, bq_row, wk_row, bk_row, ident)

    d2d = _make_sc_segnorm(rows_pad, rpt, b_pad)(e_pad, seg2d, et2d)

    out = pl.pallas_call(
        _tc2_body,
        grid=grid,
        in_specs=[_row_spec(BN, f), _row_spec(rb, LANE), _row_spec(rb, LANE),
                  _full_spec((f, f)), _full_spec((1, f)), _full_spec((1, f)),
                  _full_spec((1, f)), _full_spec((f, f)), _full_spec((1, f)),
                  _full_spec((f, f)), _full_spec((1, f)), _full_spec((f, f))],
        out_specs=_row_spec(BN, f),
        out_shape=jax.ShapeDtypeStruct((n, f), jnp.float32),
    )(x, e2d, d2d, wq_t, bq_row, wk_row, bk_row, w1_t, b1_row,
      w2_t, b2_row, ident)
    return out


# speedup vs baseline: 5.7569x; 1.1398x over previous
import functools

import jax
import jax.numpy as jnp
from jax import lax
from jax.experimental import pallas as pl
from jax.experimental.pallas import tpu as pltpu
from jax.experimental.pallas import tpu_sc as plsc

LANE = 128
NSUB = 16
VEC = 16
BN = 2048


def _softplus(v):
    return jnp.maximum(v, 0.0) + jnp.log1p(jnp.exp(-jnp.abs(v)))


_LN2 = 0.6931471805599453


def _silu(v):
    return v / (1.0 + jnp.exp(-v))


def _make_sc_gather(rows_pad, rpt, b_pad):
    mesh = plsc.VectorSubcoreMesh(core_axis_name="c", subcore_axis_name="s",
                                  num_cores=1)

    @functools.partial(
        pl.kernel,
        out_type=jax.ShapeDtypeStruct((rows_pad, LANE), jnp.float32),
        mesh=mesh,
        scratch_types=[
            pltpu.VMEM((rpt, LANE), jnp.int32),
            pltpu.VMEM((rpt, LANE), jnp.float32),
            pltpu.SemaphoreType.DMA,
        ],
    )
    def sc_gather(e_hbm, seg_hbm, out_hbm, idx_v, val_v, sem):
        wid = lax.axis_index("s")
        r0 = wid * rpt
        pltpu.sync_copy(seg_hbm.at[pl.ds(r0, rpt)], idx_v)

        def fire(j, c):
            pltpu.async_copy(e_hbm.at[idx_v.at[j]], val_v.at[j], sem)
            return c

        lax.fori_loop(0, rpt, fire, 0)

        def drain(j, c):
            pltpu.make_async_copy(e_hbm.at[idx_v.at[j]], val_v.at[j],
                                  sem).wait()
            return c

        lax.fori_loop(0, rpt, drain, 0)
        pltpu.sync_copy(val_v, out_hbm.at[pl.ds(r0, rpt)])

    return sc_gather


def _make_sc_segnorm(rows_pad, rpt, b_pad):
    spt = b_pad // NSUB
    mesh = plsc.VectorSubcoreMesh(core_axis_name="c", subcore_axis_name="s",
                                  num_cores=1)

    @functools.partial(
        pl.kernel,
        out_type=jax.ShapeDtypeStruct((rows_pad, LANE), jnp.float32),
        mesh=mesh,
        scratch_types=[
            pltpu.VMEM((rpt, LANE), jnp.int32),
            pltpu.VMEM((rpt, LANE), jnp.float32),
            pltpu.VMEM((rpt, LANE), jnp.float32),
            pltpu.VMEM((LANE,), jnp.float32),
            pltpu.VMEM((b_pad // NSUB,), jnp.float32),
            pltpu.VMEM((b_pad // NSUB,), jnp.float32),
            pltpu.VMEM((b_pad // NSUB,), jnp.float32),
            pltpu.VMEM((b_pad // NSUB,), jnp.float32),
            pltpu.VMEM_SHARED((b_pad,), jnp.float32),
            pltpu.VMEM_SHARED((b_pad,), jnp.float32),
            pltpu.VMEM_SHARED((b_pad,), jnp.float32),
            pltpu.SemaphoreType.DMA,
        ],
    )
    def sc_segnorm(e_hbm, seg_hbm, et_hbm, out_hbm, idx_v, et_v, d_v, ones_v,
                   zero_v, pred_v, cnt_v, eseg_v, pred_sh, cnt_sh, d_sh, sem):
        wid = lax.axis_index("s")
        r0 = wid * rpt
        b0 = wid * spt

        pltpu.sync_copy(seg_hbm.at[pl.ds(r0, rpt)], idx_v)
        pltpu.sync_copy(et_hbm.at[pl.ds(r0, rpt)], et_v)

        for i in range(LANE // VEC):
            ones_v[pl.ds(i * VEC, VEC)] = jnp.full((VEC,), 1.0, jnp.float32)
        for i in range(spt // VEC):
            zero_v[pl.ds(i * VEC, VEC)] = jnp.zeros((VEC,), jnp.float32)
        pltpu.sync_copy(zero_v, pred_sh.at[pl.ds(b0, spt)])
        pltpu.sync_copy(zero_v, cnt_sh.at[pl.ds(b0, spt)])
        plsc.subcore_barrier()

        def fire(j, c):
            pltpu.async_copy(et_v.at[j], pred_sh.at[idx_v.at[j]], sem,
                             add=True)
            pltpu.async_copy(ones_v, cnt_sh.at[idx_v.at[j]], sem, add=True)
            return c

        lax.fori_loop(0, rpt, fire, 0)

        def drain(j, c):
            pltpu.make_async_copy(et_v.at[j], pred_sh.at[idx_v.at[j]],
                                  sem).wait()
            pltpu.make_async_copy(ones_v, cnt_sh.at[idx_v.at[j]],
                                  sem).wait()
            return c

        lax.fori_loop(0, rpt, drain, 0)
        plsc.subcore_barrier()

        pltpu.sync_copy(pred_sh.at[pl.ds(b0, spt)], pred_v)
        pltpu.sync_copy(cnt_sh.at[pl.ds(b0, spt)], cnt_v)
        pltpu.sync_copy(e_hbm.at[pl.ds(b0, spt)], eseg_v)
        for i in range(spt // VEC):
            sl = pl.ds(i * VEC, VEC)
            pred_v[sl] = (eseg_v[sl] - pred_v[sl]) / cnt_v[sl]
        pltpu.sync_copy(pred_v, d_sh.at[pl.ds(b0, spt)])
        plsc.subcore_barrier()

        def fire_g(j, c):
            pltpu.async_copy(d_sh.at[idx_v.at[j]], d_v.at[j], sem)
            return c

        lax.fori_loop(0, rpt, fire_g, 0)

        def drain_g(j, c):
            pltpu.make_async_copy(d_sh.at[idx_v.at[j]], d_v.at[j], sem).wait()
            return c

        lax.fori_loop(0, rpt, drain_g, 0)
        pltpu.sync_copy(d_v, out_hbm.at[pl.ds(r0, rpt)])

    return sc_segnorm


def _lane_to_col(v2d, ident):
    return jnp.sum(v2d[:, None, :] * ident[None, :, :], axis=2,
                   keepdims=True)


def _col_to_lane(c3, ident):
    return jnp.sum(c3 * ident[None, :, :], axis=1)


def _tc1_body(x_ref, e_ref, wq_ref, bq_ref, wk_ref, bk_ref, id_ref,
              out_ref, emb_ref):
    f = x_ref.shape[1]
    r = x_ref.shape[0] // f
    q = jnp.dot(x_ref[...], wq_ref[...],
                preferred_element_type=jnp.float32) + bq_ref[...]
    q3 = q.reshape(r, f, f)
    e_col = _lane_to_col(e_ref[...], id_ref[...])
    k3 = jnp.abs(e_col) * wk_ref[...].reshape(1, 1, f) \
        + bk_ref[...].reshape(1, 1, f)
    sp = _softplus(k3 * q3)
    emb_ref[...] = (sp.reshape(r * f, f) - _LN2).astype(jnp.bfloat16)
    et_col = jnp.sum(sp, axis=2, keepdims=True)
    out_ref[...] = _col_to_lane(et_col, id_ref[...])


def _tc2_body(emb_ref, d_ref, w1_ref, b1_ref, w2_ref, b2_ref, id_ref,
              out_ref):
    f = emb_ref.shape[1]
    r = emb_ref.shape[0] // f
    d_col = _lane_to_col(d_ref[...], id_ref[...])
    embed = emb_ref[...].astype(jnp.float32) + _LN2
    scale = embed.reshape(r, f, f) + d_col * (1.0 / f)
    scale = scale.reshape(r * f, f)
    h = jnp.dot(_silu(scale), w1_ref[...],
                preferred_element_type=jnp.float32) + b1_ref[...]
    h = jnp.dot(_silu(h), w2_ref[...],
                preferred_element_type=jnp.float32) + b2_ref[...]
    out_ref[...] = scale + h


def _row_spec(bn, w):
    return pl.BlockSpec((bn, w), lambda i: (i, 0))


def _full_spec(shape):
    return pl.BlockSpec(shape, lambda i: (0, 0))


def kernel(x, E, num_batch, batch_seg, W_q, b_q, W_k, b_k, W_r1, b_r1,
           W_r2, b_r2, eps):
    n, f = x.shape
    b = E.shape[0]

    rows = -(-n // LANE)
    rpt = -(--(-rows // NSUB) // 8) * 8
    rows_pad = rpt * NSUB
    n_pad = rows_pad * LANE
    b_pad = -(-(b + 1) // (NSUB * VEC)) * (NSUB * VEC)

    e32 = E.astype(jnp.float32)
    e_pad = jnp.pad(e32, (0, b_pad - b))
    seg2d = jnp.pad(batch_seg.astype(jnp.int32), (0, n_pad - n),
                    constant_values=b).reshape(rows_pad, LANE)

    wq_t = W_q.T
    bq_row = b_q.reshape(1, f)
    wk_row = W_k.reshape(1, f)
    bk_row = b_k.reshape(1, f)
    w1_t = W_r1.T
    b1_row = b_r1.reshape(1, f)
    w2_t = W_r2.T
    b2_row = b_r2.reshape(1, f)

    ident = jnp.eye(f, dtype=jnp.float32)
    rb = BN // LANE

    e2d = _make_sc_gather(rows_pad, rpt, b_pad)(e_pad, seg2d)

    grid = (-(-n // BN),)
    et2d, emb16 = pl.pallas_call(
        _tc1_body,
        grid=grid,
        in_specs=[_row_spec(BN, f), _row_spec(rb, LANE), _full_spec((f, f)),
                  _full_spec((1, f)), _full_spec((1, f)), _full_spec((1, f)),
                  _full_spec((f, f))],
        out_specs=[_row_spec(rb, LANE), _row_spec(BN, f)],
        out_shape=[jax.ShapeDtypeStruct((rows_pad, LANE), jnp.float32),
                   jax.ShapeDtypeStruct((n, f), jnp.bfloat16)],
    )(x, e2d, wq_t, bq_row, wk_row, bk_row, ident)

    d2d = _make_sc_segnorm(rows_pad, rpt, b_pad)(e_pad, seg2d, et2d)

    out = pl.pallas_call(
        _tc2_body,
        grid=grid,
        in_specs=[_row_spec(BN, f), _row_spec(rb, LANE),
                  _full_spec((f, f)), _full_spec((1, f)),
                  _full_spec((f, f)), _full_spec((1, f)), _full_spec((f, f))],
        out_specs=_row_spec(BN, f),
        out_shape=jax.ShapeDtypeStruct((n, f), jnp.float32),
    )(emb16, d2d, w1_t, b1_row, w2_t, b2_row, ident)
    return out


# speedup vs baseline: 5.7588x; 1.0003x over previous
import functools

import jax
import jax.numpy as jnp
from jax import lax
from jax.experimental import pallas as pl
from jax.experimental.pallas import tpu as pltpu
from jax.experimental.pallas import tpu_sc as plsc

LANE = 128
NSUB = 16
VEC = 16
BN = 2048


def _softplus(v):
    return jnp.maximum(v, 0.0) + jnp.log1p(jnp.exp(-jnp.abs(v)))


_LN2 = 0.6931471805599453


def _silu(v):
    return v / (1.0 + jnp.exp(-v))


def _make_sc_gather(rows_pad, rpt, b_pad):
    mesh = plsc.VectorSubcoreMesh(core_axis_name="c", subcore_axis_name="s",
                                  num_cores=1)

    @functools.partial(
        pl.kernel,
        out_type=jax.ShapeDtypeStruct((rows_pad, LANE), jnp.float32),
        mesh=mesh,
        scratch_types=[
            pltpu.VMEM((rpt, LANE), jnp.int32),
            pltpu.VMEM((rpt, LANE), jnp.float32),
            pltpu.SemaphoreType.DMA,
        ],
    )
    def sc_gather(e_hbm, seg_hbm, out_hbm, idx_v, val_v, sem):
        wid = lax.axis_index("s")
        r0 = wid * rpt
        pltpu.sync_copy(seg_hbm.at[pl.ds(r0, rpt)], idx_v)

        def fire(j, c):
            pltpu.async_copy(e_hbm.at[idx_v.at[j]], val_v.at[j], sem)
            return c

        lax.fori_loop(0, rpt, fire, 0)
        pltpu.make_async_copy(out_hbm.at[pl.ds(r0, rpt)], val_v, sem).wait()
        pltpu.sync_copy(val_v, out_hbm.at[pl.ds(r0, rpt)])

    return sc_gather


def _make_sc_segnorm(rows_pad, rpt, b_pad):
    spt = b_pad // NSUB
    mesh = plsc.VectorSubcoreMesh(core_axis_name="c", subcore_axis_name="s",
                                  num_cores=1)

    @functools.partial(
        pl.kernel,
        out_type=jax.ShapeDtypeStruct((rows_pad, LANE), jnp.float32),
        mesh=mesh,
        scratch_types=[
            pltpu.VMEM((rpt, LANE), jnp.int32),
            pltpu.VMEM((rpt, LANE), jnp.float32),
            pltpu.VMEM((rpt, LANE), jnp.float32),
            pltpu.VMEM((LANE,), jnp.float32),
            pltpu.VMEM((b_pad // NSUB,), jnp.float32),
            pltpu.VMEM((b_pad // NSUB,), jnp.float32),
            pltpu.VMEM((b_pad // NSUB,), jnp.float32),
            pltpu.VMEM((b_pad // NSUB,), jnp.float32),
            pltpu.VMEM_SHARED((b_pad,), jnp.float32),
            pltpu.VMEM_SHARED((b_pad,), jnp.float32),
            pltpu.VMEM_SHARED((b_pad,), jnp.float32),
            pltpu.SemaphoreType.DMA,
        ],
    )
    def sc_segnorm(e_hbm, seg_hbm, et_hbm, out_hbm, idx_v, et_v, d_v, ones_v,
                   zero_v, pred_v, cnt_v, eseg_v, pred_sh, cnt_sh,
                   d_sh, sem):
        wid = lax.axis_index("s")
        r0 = wid * rpt
        b0 = wid * spt

        pltpu.sync_copy(seg_hbm.at[pl.ds(r0, rpt)], idx_v)
        pltpu.sync_copy(et_hbm.at[pl.ds(r0, rpt)], et_v)

        for i in range(LANE // VEC):
            ones_v[pl.ds(i * VEC, VEC)] = jnp.full((VEC,), 1.0, jnp.float32)
        for i in range(spt // VEC):
            zero_v[pl.ds(i * VEC, VEC)] = jnp.zeros((VEC,), jnp.float32)
        pltpu.sync_copy(zero_v, pred_sh.at[pl.ds(b0, spt)])
        pltpu.sync_copy(zero_v, cnt_sh.at[pl.ds(b0, spt)])
        plsc.subcore_barrier()

        def fire(j, c):
            pltpu.async_copy(et_v.at[j], pred_sh.at[idx_v.at[j]], sem,
                             add=True)
            pltpu.async_copy(ones_v, cnt_sh.at[idx_v.at[j]], sem, add=True)
            return c

        lax.fori_loop(0, rpt, fire, 0)

        pltpu.make_async_copy(et_hbm.at[pl.ds(r0, rpt)], et_v, sem).wait()
        pltpu.make_async_copy(et_hbm.at[pl.ds(r0, rpt)], et_v, sem).wait()
        plsc.subcore_barrier()

        pltpu.sync_copy(pred_sh.at[pl.ds(b0, spt)], pred_v)
        pltpu.sync_copy(cnt_sh.at[pl.ds(b0, spt)], cnt_v)
        pltpu.sync_copy(e_hbm.at[pl.ds(b0, spt)], eseg_v)
        for i in range(spt // VEC):
            sl = pl.ds(i * VEC, VEC)
            pred_v[sl] = (eseg_v[sl] - pred_v[sl]) / cnt_v[sl]
        pltpu.sync_copy(pred_v, d_sh.at[pl.ds(b0, spt)])
        plsc.subcore_barrier()

        def fire_g(j, c):
            pltpu.async_copy(d_sh.at[idx_v.at[j]], d_v.at[j], sem)
            return c

        lax.fori_loop(0, rpt, fire_g, 0)
        pltpu.make_async_copy(et_hbm.at[pl.ds(r0, rpt)], d_v, sem).wait()
        pltpu.sync_copy(d_v, out_hbm.at[pl.ds(r0, rpt)])

    return sc_segnorm


def _lane_to_col(v2d, ident):
    return jnp.sum(v2d[:, None, :] * ident[None, :, :], axis=2,
                   keepdims=True)


def _col_to_lane(c3, ident):
    return jnp.sum(c3 * ident[None, :, :], axis=1)


def _tc1_body(x_ref, e_ref, wq_ref, bq_ref, wk_ref, bk_ref, id_ref,
              out_ref, emb_ref):
    f = x_ref.shape[1]
    r = x_ref.shape[0] // f
    q = jnp.dot(x_ref[...], wq_ref[...],
                preferred_element_type=jnp.float32) + bq_ref[...]
    q3 = q.reshape(r, f, f)
    e_col = _lane_to_col(e_ref[...], id_ref[...])
    k3 = jnp.abs(e_col) * wk_ref[...].reshape(1, 1, f) \
        + bk_ref[...].reshape(1, 1, f)
    sp = _softplus(k3 * q3)
    emb_ref[...] = (sp.reshape(r * f, f) - _LN2).astype(jnp.bfloat16)
    et_col = jnp.sum(sp, axis=2, keepdims=True)
    out_ref[...] = _col_to_lane(et_col, id_ref[...])


def _tc2_body(emb_ref, d_ref, w1_ref, b1_ref, w2_ref, b2_ref, id_ref,
              out_ref):
    f = emb_ref.shape[1]
    r = emb_ref.shape[0] // f
    d_col = _lane_to_col(d_ref[...], id_ref[...])
    embed = emb_ref[...].astype(jnp.float32) + _LN2
    scale = embed.reshape(r, f, f) + d_col * (1.0 / f)
    scale = scale.reshape(r * f, f)
    h = jnp.dot(_silu(scale), w1_ref[...],
                preferred_element_type=jnp.float32) + b1_ref[...]
    h = jnp.dot(_silu(h), w2_ref[...],
                preferred_element_type=jnp.float32) + b2_ref[...]
    out_ref[...] = scale + h


def _row_spec(bn, w):
    return pl.BlockSpec((bn, w), lambda i: (i, 0))


def _full_spec(shape):
    return pl.BlockSpec(shape, lambda i: (0, 0))


def kernel(x, E, num_batch, batch_seg, W_q, b_q, W_k, b_k, W_r1, b_r1,
           W_r2, b_r2, eps):
    n, f = x.shape
    b = E.shape[0]

    rows = -(-n // LANE)
    rpt = -(--(-rows // NSUB) // 8) * 8
    rows_pad = rpt * NSUB
    n_pad = rows_pad * LANE
    b_pad = -(-(b + 1) // (NSUB * VEC)) * (NSUB * VEC)

    e32 = E.astype(jnp.float32)
    e_pad = jnp.pad(e32, (0, b_pad - b))
    seg2d = jnp.pad(batch_seg.astype(jnp.int32), (0, n_pad - n),
                    constant_values=b).reshape(rows_pad, LANE)

    wq_t = W_q.T
    bq_row = b_q.reshape(1, f)
    wk_row = W_k.reshape(1, f)
    bk_row = b_k.reshape(1, f)
    w1_t = W_r1.T
    b1_row = b_r1.reshape(1, f)
    w2_t = W_r2.T
    b2_row = b_r2.reshape(1, f)

    ident = jnp.eye(f, dtype=jnp.float32)
    rb = BN // LANE

    e2d = _make_sc_gather(rows_pad, rpt, b_pad)(e_pad, seg2d)

    grid = (-(-n // BN),)
    et2d, emb16 = pl.pallas_call(
        _tc1_body,
        grid=grid,
        in_specs=[_row_spec(BN, f), _row_spec(rb, LANE), _full_spec((f, f)),
                  _full_spec((1, f)), _full_spec((1, f)), _full_spec((1, f)),
                  _full_spec((f, f))],
        out_specs=[_row_spec(rb, LANE), _row_spec(BN, f)],
        out_shape=[jax.ShapeDtypeStruct((rows_pad, LANE), jnp.float32),
                   jax.ShapeDtypeStruct((n, f), jnp.bfloat16)],
    )(x, e2d, wq_t, bq_row, wk_row, bk_row, ident)

    d2d = _make_sc_segnorm(rows_pad, rpt, b_pad)(e_pad, seg2d, et2d)

    out = pl.pallas_call(
        _tc2_body,
        grid=grid,
        in_specs=[_row_spec(BN, f), _row_spec(rb, LANE),
                  _full_spec((f, f)), _full_spec((1, f)),
                  _full_spec((f, f)), _full_spec((1, f)), _full_spec((f, f))],
        out_specs=_row_spec(BN, f),
        out_shape=jax.ShapeDtypeStruct((n, f), jnp.float32),
    )(emb16, d2d, w1_t, b1_row, w2_t, b2_row, ident)
    return out
